# baseline (speedup 1.0000x reference)
import jax
import jax.numpy as jnp
from jax.experimental import pallas as pl
from jax.experimental.pallas import tpu as pltpu


D_MODEL = 4096
NUM_EXPERTS = 64
TILE_TOK = 1024


def _router_kernel(x_ref, w_ref, b_ref, out_hbm, obuf, osem):
    i = pl.program_id(0)
    n = pl.num_programs(0)
    s = i % 2

    def _copy(step, slot):
        return pltpu.make_async_copy(
            obuf.at[slot],
            out_hbm.at[pl.ds(step * TILE_TOK, TILE_TOK), :],
            osem.at[slot],
        )

    @pl.when(i >= 2)
    def _wait_prev():
        _copy(i - 2, s).wait()

    logits = jax.lax.dot_general(
        x_ref[...], w_ref[...],
        dimension_numbers=(((1,), (1,)), ((), ())),
        preferred_element_type=jnp.float32,
    )
    logits = logits + b_ref[...]
    m = jnp.max(logits, axis=-1, keepdims=True)
    e = jnp.exp(logits - m)
    obuf[s] = e / jnp.sum(e, axis=-1, keepdims=True)
    _copy(i, s).start()

    @pl.when(i == n - 1)
    def _drain():
        _copy(i - 1, (s + 1) % 2).wait()
        _copy(i, s).wait()


def kernel(x, W, b):
    n_tok = x.shape[0]
    grid = (n_tok // TILE_TOK,)
    return pl.pallas_call(
        _router_kernel,
        grid=grid,
        in_specs=[
            pl.BlockSpec((TILE_TOK, D_MODEL), lambda i: (i, 0)),
            pl.BlockSpec((NUM_EXPERTS, D_MODEL), lambda i: (0, 0)),
            pl.BlockSpec((NUM_EXPERTS,), lambda i: (0,)),
        ],
        out_specs=pl.BlockSpec(memory_space=pltpu.MemorySpace.HBM),
        out_shape=jax.ShapeDtypeStruct((n_tok, NUM_EXPERTS), jnp.float32),
        scratch_shapes=[
            pltpu.VMEM((2, TILE_TOK, NUM_EXPERTS), jnp.float32),
            pltpu.SemaphoreType.DMA((2,)),
        ],
        compiler_params=pltpu.CompilerParams(
            dimension_semantics=("arbitrary",),
        ),
    )(x, W, b)


# speedup vs baseline: 1.0086x; 1.0086x over previous
import jax
import jax.numpy as jnp
from jax.experimental import pallas as pl
from jax.experimental.pallas import tpu as pltpu


D_MODEL = 4096
NUM_EXPERTS = 64
TILE_TOK = 1024
GROUP = 4
GROUP_ROWS = GROUP * TILE_TOK


def _router_kernel(x_ref, w_ref, b_ref, out_hbm, obuf, osem):
    i = pl.program_id(0)
    n = pl.num_programs(0)
    g = i // GROUP
    slot = g % 2
    off = (i % GROUP) * TILE_TOK

    def _copy(group, s):
        return pltpu.make_async_copy(
            obuf.at[s],
            out_hbm.at[pl.ds(group * GROUP_ROWS, GROUP_ROWS), :],
            osem.at[s],
        )

    @pl.when((i % GROUP == 0) & (g >= 2))
    def _wait_prev():
        _copy(g - 2, slot).wait()

    logits = jax.lax.dot_general(
        x_ref[...], w_ref[...],
        dimension_numbers=(((1,), (1,)), ((), ())),
        preferred_element_type=jnp.float32,
    )
    logits = logits + b_ref[...]
    m = jnp.max(logits, axis=-1, keepdims=True)
    e = jnp.exp(logits - m)
    obuf[slot, pl.ds(off, TILE_TOK), :] = e / jnp.sum(e, axis=-1, keepdims=True)

    @pl.when(i % GROUP == GROUP - 1)
    def _start_copy():
        _copy(g, slot).start()

    @pl.when(i == n - 1)
    def _drain():
        _copy(g - 1, (slot + 1) % 2).wait()
        _copy(g, slot).wait()


def kernel(x, W, b):
    n_tok = x.shape[0]
    grid = (n_tok // TILE_TOK,)
    return pl.pallas_call(
        _router_kernel,
        grid=grid,
        in_specs=[
            pl.BlockSpec((TILE_TOK, D_MODEL), lambda i: (i, 0)),
            pl.BlockSpec((NUM_EXPERTS, D_MODEL), lambda i: (0, 0)),
            pl.BlockSpec((NUM_EXPERTS,), lambda i: (0,)),
        ],
        out_specs=pl.BlockSpec(memory_space=pltpu.MemorySpace.HBM),
        out_shape=jax.ShapeDtypeStruct((n_tok, NUM_EXPERTS), jnp.float32),
        scratch_shapes=[
            pltpu.VMEM((2, GROUP_ROWS, NUM_EXPERTS), jnp.float32),
            pltpu.SemaphoreType.DMA((2,)),
        ],
        compiler_params=pltpu.CompilerParams(
            dimension_semantics=("arbitrary",),
        ),
    )(x, W, b)
